# baseline (speedup 1.0000x reference)
import functools

import jax
import jax.numpy as jnp
import numpy as np
from jax import lax
from jax.experimental import pallas as pl
from jax.experimental.pallas import tpu as pltpu
from jax.experimental.pallas import tpu_sc as plsc

_NE = 16
_BM = 192
_BH = 3072
_GW_PAD = 128


def _route_body(x_ref, gw_ref, gb_ref, dst_ref, te_ref, oh_s, cum_s):
    T = x_ref.shape[0]
    nb = T // 128
    logits = jnp.dot(x_ref[...], gw_ref[...], preferred_element_type=jnp.float32)
    logits = logits + gb_ref[...]
    m = jnp.max(logits, axis=-1, keepdims=True)
    ids = lax.broadcasted_iota(jnp.int32, logits.shape, 1)
    cand = jnp.where(logits == m, ids, _GW_PAD)
    eid = jnp.min(cand, axis=-1, keepdims=True)
    oh_s[...] = (ids == eid).astype(jnp.float32)

    r128 = lax.broadcasted_iota(jnp.int32, (128, 128), 0)
    c128 = lax.broadcasted_iota(jnp.int32, (128, 128), 1)
    tri = (r128 >= c128).astype(jnp.float32)

    def body(b, base):
        blk = oh_s[pl.ds(b * 128, 128), :]
        cum = jnp.dot(tri, blk, preferred_element_type=jnp.float32) + base
        cum_s[pl.ds(b * 128, 128), :] = cum
        return cum[127:128, :]

    counts = lax.fori_loop(0, nb, body, jnp.zeros((1, 128), jnp.float32))

    ntiles = jnp.floor((counts + (_BM - 1.0)) / _BM)
    le_mat = (r128 <= c128).astype(jnp.float32)
    tile_end = jnp.dot(ntiles, le_mat, preferred_element_type=jnp.float32)
    tile_start = tile_end - ntiles
    eye = (r128 == c128).astype(jnp.float32)
    te_col = jnp.sum(jnp.broadcast_to(tile_end, (128, 128)) * eye, axis=1,
                     keepdims=True)
    m2 = jnp.logical_and(te_col <= c128.astype(jnp.float32), r128 < _NE)
    te_row = jnp.sum(m2.astype(jnp.float32), axis=0, keepdims=True)
    te_ref[...] = jnp.concatenate(
        [te_row, tile_end], axis=0).astype(jnp.int32)

    rank = jnp.sum(cum_s[...] * oh_s[...], axis=1, keepdims=True) - 1.0
    sel = jnp.sum(tile_start * _BM * oh_s[...], axis=1, keepdims=True)
    dst_ref[...] = (sel + rank).astype(jnp.int32)


def _ffn_body(te_ref, x_ref, w1_ref, b1_ref, w2_ref, b2_ref, o_ref):
    h = pl.program_id(1)
    acc = jnp.dot(x_ref[...], w1_ref[0], preferred_element_type=jnp.float32)
    acc = jnp.maximum(acc + b1_ref[0], 0.0)
    y = jnp.dot(acc, w2_ref[0], preferred_element_type=jnp.float32)

    @pl.when(h == 0)
    def _():
        o_ref[...] = y + b2_ref[0]

    @pl.when(h != 0)
    def _():
        o_ref[...] += y


_NC = 2
_NS = 16
_NW = _NC * _NS


def _sc_scatter_rows(t2, dst, nslots):
    T, d = t2.shape
    bpw = T // _NW
    mesh = plsc.VectorSubcoreMesh(core_axis_name="c", subcore_axis_name="s")

    @functools.partial(
        pl.kernel, mesh=mesh,
        out_type=jax.ShapeDtypeStruct((nslots, d), jnp.float32),
        scratch_types=[
            pltpu.VMEM((bpw,), jnp.int32),
            pltpu.VMEM((bpw, d), jnp.float32),
            pltpu.SemaphoreType.DMA,
        ],
    )
    def k(t2_hbm, dst_hbm, xs_hbm, idx_v, rows_v, sem):
        wid = lax.axis_index("s") * _NC + lax.axis_index("c")
        base = wid * bpw
        pltpu.sync_copy(dst_hbm.at[pl.ds(base, bpw)], idx_v)
        pltpu.sync_copy(t2_hbm.at[pl.ds(base, bpw)], rows_v)
        pltpu.async_copy(rows_v, xs_hbm.at[idx_v], sem).wait()

    return k(t2, dst)


def _sc_gather_rows(ys, dst):
    T = dst.shape[0]
    d = ys.shape[1]
    bpw = T // _NW
    mesh = plsc.VectorSubcoreMesh(core_axis_name="c", subcore_axis_name="s")

    @functools.partial(
        pl.kernel, mesh=mesh,
        out_type=jax.ShapeDtypeStruct((T, d), jnp.float32),
        scratch_types=[
            pltpu.VMEM((bpw,), jnp.int32),
            pltpu.VMEM((bpw, d), jnp.float32),
            pltpu.SemaphoreType.DMA,
        ],
    )
    def k(ys_hbm, dst_hbm, out_hbm, idx_v, rows_v, sem):
        wid = lax.axis_index("s") * _NC + lax.axis_index("c")
        base = wid * bpw
        pltpu.sync_copy(dst_hbm.at[pl.ds(base, bpw)], idx_v)
        pltpu.async_copy(ys_hbm.at[idx_v], rows_v, sem).wait()
        pltpu.sync_copy(rows_v, out_hbm.at[pl.ds(base, bpw)])

    return k(ys, dst)


def kernel(x, gate_w, gate_b, w1, b1, w2, b2):
    fsz = x.shape[:-1]
    d = x.shape[-1]
    t2 = x.reshape(-1, d)
    T = t2.shape[0]
    hd = w1.shape[-1]
    nh = hd // _BH
    NT = min(_NE, T) + (T - min(_NE, T)) // _BM

    gw_p = jnp.zeros((d, _GW_PAD), gate_w.dtype).at[:, :_NE].set(gate_w)
    gb_p = jnp.full((1, _GW_PAD), -1e30, gate_b.dtype).at[0, :_NE].set(gate_b)
    dst2, te2 = pl.pallas_call(
        _route_body,
        out_shape=[
            jax.ShapeDtypeStruct((T, 1), jnp.int32),
            jax.ShapeDtypeStruct((2, _GW_PAD), jnp.int32),
        ],
        scratch_shapes=[
            pltpu.VMEM((T, _GW_PAD), jnp.float32),
            pltpu.VMEM((T, _GW_PAD), jnp.float32),
        ],
    )(t2, gw_p, gb_p)
    dst = dst2[:, 0]
    nt_actual = te2[1, _GW_PAD - 1]

    dst = dst.astype(jnp.int32)
    xs = _sc_scatter_rows(t2, dst, NT * _BM)

    def _te(t, te_r):
        return jnp.minimum(te_r[0, t], _NE - 1)

    grid_spec = pltpu.PrefetchScalarGridSpec(
        num_scalar_prefetch=1,
        grid=(nt_actual, nh),
        in_specs=[
            pl.BlockSpec((_BM, d), lambda t, h, te_r: (t, 0)),
            pl.BlockSpec((1, d, _BH), lambda t, h, te_r: (_te(t, te_r), 0, h)),
            pl.BlockSpec((1, 1, _BH), lambda t, h, te_r: (_te(t, te_r), 0, h)),
            pl.BlockSpec((1, _BH, d), lambda t, h, te_r: (_te(t, te_r), h, 0)),
            pl.BlockSpec((1, 1, d), lambda t, h, te_r: (_te(t, te_r), 0, 0)),
        ],
        out_specs=pl.BlockSpec((_BM, d), lambda t, h, te_r: (t, 0)),
    )
    ys = pl.pallas_call(
        _ffn_body,
        grid_spec=grid_spec,
        out_shape=jax.ShapeDtypeStruct((NT * _BM, d), jnp.float32),
    )(te2, xs, w1, b1[:, None, :], w2, b2[:, None, :])

    out = _sc_gather_rows(ys, dst)
    return out.reshape(fsz + (d,))


# speedup vs baseline: 1.0192x; 1.0192x over previous
import functools

import jax
import jax.numpy as jnp
import numpy as np
from jax import lax
from jax.experimental import pallas as pl
from jax.experimental.pallas import tpu as pltpu
from jax.experimental.pallas import tpu_sc as plsc

_NE = 16
_BM = 160
_BH = 3072
_GW_PAD = 128


def _route_body(x_ref, gw_ref, gb_ref, dst_ref, te_ref, oh_s, cum_s):
    T = x_ref.shape[0]
    nb = T // 128
    logits = jnp.dot(x_ref[...], gw_ref[...], preferred_element_type=jnp.float32)
    logits = logits + gb_ref[...]
    m = jnp.max(logits, axis=-1, keepdims=True)
    ids = lax.broadcasted_iota(jnp.int32, logits.shape, 1)
    cand = jnp.where(logits == m, ids, _GW_PAD)
    eid = jnp.min(cand, axis=-1, keepdims=True)
    oh_s[...] = (ids == eid).astype(jnp.float32)

    r128 = lax.broadcasted_iota(jnp.int32, (128, 128), 0)
    c128 = lax.broadcasted_iota(jnp.int32, (128, 128), 1)
    tri = (r128 >= c128).astype(jnp.float32)

    def body(b, base):
        blk = oh_s[pl.ds(b * 128, 128), :]
        cum = jnp.dot(tri, blk, preferred_element_type=jnp.float32) + base
        cum_s[pl.ds(b * 128, 128), :] = cum
        return cum[127:128, :]

    counts = lax.fori_loop(0, nb, body, jnp.zeros((1, 128), jnp.float32))

    ntiles = jnp.floor((counts + (_BM - 1.0)) / _BM)
    le_mat = (r128 <= c128).astype(jnp.float32)
    tile_end = jnp.dot(ntiles, le_mat, preferred_element_type=jnp.float32)
    tile_start = tile_end - ntiles
    eye = (r128 == c128).astype(jnp.float32)
    te_col = jnp.sum(jnp.broadcast_to(tile_end, (128, 128)) * eye, axis=1,
                     keepdims=True)
    m2 = jnp.logical_and(te_col <= c128.astype(jnp.float32), r128 < _NE)
    te_row = jnp.sum(m2.astype(jnp.float32), axis=0, keepdims=True)
    te_ref[...] = jnp.concatenate(
        [te_row, tile_end], axis=0).astype(jnp.int32)

    rank = jnp.sum(cum_s[...] * oh_s[...], axis=1, keepdims=True) - 1.0
    sel = jnp.sum(tile_start * _BM * oh_s[...], axis=1, keepdims=True)
    dst_ref[...] = (sel + rank).astype(jnp.int32)


def _ffn_body(te_ref, x_ref, w1_ref, b1_ref, w2_ref, b2_ref, o_ref):
    h = pl.program_id(1)
    acc = jnp.dot(x_ref[...], w1_ref[0], preferred_element_type=jnp.float32)
    acc = jnp.maximum(acc + b1_ref[0], 0.0)
    y = jnp.dot(acc, w2_ref[0], preferred_element_type=jnp.float32)

    @pl.when(h == 0)
    def _():
        o_ref[...] = y + b2_ref[0]

    @pl.when(h != 0)
    def _():
        o_ref[...] += y


_NC = 2
_NS = 16
_NW = _NC * _NS


def _sc_scatter_rows(t2, dst, nslots):
    T, d = t2.shape
    bpw = T // _NW
    mesh = plsc.VectorSubcoreMesh(core_axis_name="c", subcore_axis_name="s")

    @functools.partial(
        pl.kernel, mesh=mesh,
        out_type=jax.ShapeDtypeStruct((nslots, d), jnp.float32),
        scratch_types=[
            pltpu.VMEM((bpw,), jnp.int32),
            pltpu.VMEM((bpw, d), jnp.float32),
            pltpu.SemaphoreType.DMA,
        ],
    )
    def k(t2_hbm, dst_hbm, xs_hbm, idx_v, rows_v, sem):
        wid = lax.axis_index("s") * _NC + lax.axis_index("c")
        base = wid * bpw
        pltpu.sync_copy(dst_hbm.at[pl.ds(base, bpw)], idx_v)
        pltpu.sync_copy(t2_hbm.at[pl.ds(base, bpw)], rows_v)
        pltpu.async_copy(rows_v, xs_hbm.at[idx_v], sem).wait()

    return k(t2, dst)


def _sc_gather_rows(ys, dst):
    T = dst.shape[0]
    d = ys.shape[1]
    bpw = T // _NW
    mesh = plsc.VectorSubcoreMesh(core_axis_name="c", subcore_axis_name="s")

    @functools.partial(
        pl.kernel, mesh=mesh,
        out_type=jax.ShapeDtypeStruct((T, d), jnp.float32),
        scratch_types=[
            pltpu.VMEM((bpw,), jnp.int32),
            pltpu.VMEM((bpw, d), jnp.float32),
            pltpu.SemaphoreType.DMA,
        ],
    )
    def k(ys_hbm, dst_hbm, out_hbm, idx_v, rows_v, sem):
        wid = lax.axis_index("s") * _NC + lax.axis_index("c")
        base = wid * bpw
        pltpu.sync_copy(dst_hbm.at[pl.ds(base, bpw)], idx_v)
        pltpu.async_copy(ys_hbm.at[idx_v], rows_v, sem).wait()
        pltpu.sync_copy(rows_v, out_hbm.at[pl.ds(base, bpw)])

    return k(ys, dst)


def kernel(x, gate_w, gate_b, w1, b1, w2, b2):
    fsz = x.shape[:-1]
    d = x.shape[-1]
    t2 = x.reshape(-1, d)
    T = t2.shape[0]
    hd = w1.shape[-1]
    nh = hd // _BH
    NT = min(_NE, T) + (T - min(_NE, T)) // _BM

    gw_p = jnp.zeros((d, _GW_PAD), gate_w.dtype).at[:, :_NE].set(gate_w)
    gb_p = jnp.full((1, _GW_PAD), -1e30, gate_b.dtype).at[0, :_NE].set(gate_b)
    dst2, te2 = pl.pallas_call(
        _route_body,
        out_shape=[
            jax.ShapeDtypeStruct((T, 1), jnp.int32),
            jax.ShapeDtypeStruct((2, _GW_PAD), jnp.int32),
        ],
        scratch_shapes=[
            pltpu.VMEM((T, _GW_PAD), jnp.float32),
            pltpu.VMEM((T, _GW_PAD), jnp.float32),
        ],
    )(t2, gw_p, gb_p)
    dst = dst2[:, 0]
    nt_actual = te2[1, _GW_PAD - 1]

    dst = dst.astype(jnp.int32)
    xs = _sc_scatter_rows(t2, dst, NT * _BM)

    def _te(t, te_r):
        return jnp.minimum(te_r[0, t], _NE - 1)

    grid_spec = pltpu.PrefetchScalarGridSpec(
        num_scalar_prefetch=1,
        grid=(nt_actual, nh),
        in_specs=[
            pl.BlockSpec((_BM, d), lambda t, h, te_r: (t, 0)),
            pl.BlockSpec((1, d, _BH), lambda t, h, te_r: (_te(t, te_r), 0, h)),
            pl.BlockSpec((1, 1, _BH), lambda t, h, te_r: (_te(t, te_r), 0, h)),
            pl.BlockSpec((1, _BH, d), lambda t, h, te_r: (_te(t, te_r), h, 0)),
            pl.BlockSpec((1, 1, d), lambda t, h, te_r: (_te(t, te_r), 0, 0)),
        ],
        out_specs=pl.BlockSpec((_BM, d), lambda t, h, te_r: (t, 0)),
    )
    ys = pl.pallas_call(
        _ffn_body,
        grid_spec=grid_spec,
        out_shape=jax.ShapeDtypeStruct((NT * _BM, d), jnp.float32),
    )(te2, xs, w1, b1[:, None, :], w2, b2[:, None, :])

    out = _sc_gather_rows(ys, dst)
    return out.reshape(fsz + (d,))
